# baseline (speedup 1.0000x reference)
import functools

import jax
import jax.numpy as jnp
from jax import lax
from jax.experimental import pallas as pl
from jax.experimental.pallas import tpu as pltpu
from jax.experimental.pallas import tpu_sc as plsc

_HIDDEN = 4096
_E = 64
_K = 8
_T = 8192
_BT = 512
_NB = _T // _BT

_NW = 32
_RPW = _T // _NW
_BROWS = _RPW // 2
_OROWS = _RPW * _K // 128


def _matmul_kernel(x_ref, w_ref, logits_ref):
    logits_ref[...] = jax.lax.dot_general(
        x_ref[...], w_ref[...],
        dimension_numbers=(((1,), (1,)), ((), ())),
        preferred_element_type=jnp.float32,
    )


_sc_mesh = plsc.VectorSubcoreMesh(core_axis_name="c", subcore_axis_name="s")


@functools.partial(
    pl.kernel,
    mesh=_sc_mesh,
    out_type=[
        jax.ShapeDtypeStruct((_T * _K // 128, 128), jnp.float32),
        jax.ShapeDtypeStruct((_T * _K // 128, 128), jnp.int32),
    ],
    scratch_types=[
        pltpu.VMEM((_BROWS, 128), jnp.float32),
        pltpu.VMEM((_OROWS, 128), jnp.float32),
        pltpu.VMEM((_OROWS, 128), jnp.int32),
    ],
)
def _sc_topk(logits_hbm, topw_hbm, topi_hbm, buf, ow, oi):
    wid = lax.axis_index("s") * 2 + lax.axis_index("c")
    pltpu.sync_copy(logits_hbm.at[pl.ds(wid * _BROWS, _BROWS), :], buf)
    iota = lax.iota(jnp.int32, 16)
    perms = [jnp.bitwise_xor(iota, sh) for sh in (8, 4, 2, 1)]

    gdn = lax.GatherDimensionNumbers(
        offset_dims=(), collapsed_slice_dims=(0,), start_index_map=(0,))

    def permute(v, p):
        return lax.gather(
            v, p[:, None], gdn, slice_sizes=(1,),
            mode=lax.GatherScatterMode.PROMISE_IN_BOUNDS)

    def allreduce(v, op):
        for p in perms:
            v = op(v, permute(v, p))
        return v

    def row_topk(vs):
        m = vs[0]
        for v in vs[1:]:
            m = jnp.maximum(m, v)
        mrow = allreduce(m, jnp.maximum)
        es = [jnp.exp(v - mrow) for v in vs]
        ssum = es[0]
        for e in es[1:]:
            ssum = ssum + e
        s = allreduce(ssum, jnp.add)
        ws = [e / s for e in es]
        tvals = []
        tidx = []
        total = jnp.zeros((16,), jnp.float32)
        for _ in range(_K):
            m4 = ws[0]
            for w in ws[1:]:
                m4 = jnp.maximum(m4, w)
            mv = allreduce(m4, jnp.maximum)
            ii = jnp.where(ws[0] == mv, iota, _E)
            for j in range(1, len(ws)):
                ii = jnp.minimum(
                    ii, jnp.where(ws[j] == mv, iota + 16 * j, _E))
            ix = allreduce(ii, jnp.minimum)
            tvals.append(mv)
            tidx.append(ix)
            total = total + mv
            ws = [jnp.where((iota + 16 * j) == ix, -jnp.inf, w)
                  for j, w in enumerate(ws)]
        return tvals, tidx, total

    def orow_body(p2, carry):
        for q in range(8):
            p = p2 * 8 + q
            wvec = jnp.zeros((16,), jnp.float32)
            ivec = jnp.zeros((16,), jnp.int32)
            for half in range(2):
                vs = [buf[p, pl.ds(64 * half + 16 * j, 16)]
                      for j in range(4)]
                tvals, tidx, total = row_topk(vs)
                for k in range(_K):
                    lane = half * _K + k
                    wvec = jnp.where(iota == lane, tvals[k] / total, wvec)
                    ivec = jnp.where(iota == lane, tidx[k], ivec)
            ow[p2, pl.ds(16 * q, 16)] = wvec
            oi[p2, pl.ds(16 * q, 16)] = ivec
        return carry

    lax.fori_loop(0, _OROWS, orow_body, 0)

    pltpu.sync_copy(ow, topw_hbm.at[pl.ds(wid * _OROWS, _OROWS), :])
    pltpu.sync_copy(oi, topi_hbm.at[pl.ds(wid * _OROWS, _OROWS), :])


@jax.jit
def kernel(x, W_gate):
    logits = pl.pallas_call(
        _matmul_kernel,
        grid=(_NB,),
        in_specs=[
            pl.BlockSpec((_BT, _HIDDEN), lambda i: (i, 0)),
            pl.BlockSpec((_E, _HIDDEN), lambda i: (0, 0)),
        ],
        out_specs=pl.BlockSpec((_BT, _E), lambda i: (i, 0)),
        out_shape=jax.ShapeDtypeStruct((_T, _E), jnp.float32),
    )(x, W_gate)
    logits128 = logits.reshape(_T // 2, 128)
    topw2, topi2 = _sc_topk(logits128)
    topw = topw2.reshape(_T, _K)
    topi = topi2.reshape(_T, _K)
    return topw, topi, logits


# speedup vs baseline: 2.4442x; 2.4442x over previous
import jax
import jax.numpy as jnp
from jax.experimental import pallas as pl
from jax.experimental.pallas import tpu as pltpu

_HIDDEN = 4096
_E = 64
_K = 8
_BT = 1024
_NB = 8192 // _BT
_RC = 64


def _topk_block(logits, topw_ref, topi_ref):
    iota = jax.lax.broadcasted_iota(
        jnp.int32, (_RC, _E), 1).astype(jnp.float32)
    for c in range(_BT // _RC):
        l = logits[c * _RC:(c + 1) * _RC, :]
        m = jnp.max(l, axis=1, keepdims=True)
        e = jnp.exp(l - m)
        s = jnp.sum(e, axis=1, keepdims=True)
        vals = e / s
        tops = []
        idxs = []
        total = jnp.zeros((_RC, 1), jnp.float32)
        for _ in range(_K):
            mv = jnp.max(vals, axis=1, keepdims=True)
            ix = jnp.min(jnp.where(vals == mv, iota, float(_E)),
                         axis=1, keepdims=True)
            tops.append(mv)
            idxs.append(ix)
            total = total + mv
            vals = jnp.where(iota == ix, -jnp.inf, vals)
        for j in range(_K):
            topw_ref[c * _RC:(c + 1) * _RC, j:j + 1] = tops[j] / total
            topi_ref[c * _RC:(c + 1) * _RC, j:j + 1] = (
                idxs[j].astype(jnp.int32))


def _router_kernel(x_ref, w_ref, topw_ref, topi_ref, logits_ref, lbuf):
    i = pl.program_id(0)

    logits = jax.lax.dot_general(
        x_ref[...], w_ref[...],
        dimension_numbers=(((1,), (1,)), ((), ())),
        preferred_element_type=jnp.float32,
    )
    logits_ref[...] = logits

    prev = lbuf[(i - 1) % 2]
    _topk_block(prev, topw_ref, topi_ref)
    lbuf[i % 2] = logits


@jax.jit
def kernel(x, W_gate):
    tokens = x.shape[0]
    topw, topi, logits = pl.pallas_call(
        _router_kernel,
        grid=(_NB + 1,),
        in_specs=[
            pl.BlockSpec((_BT, _HIDDEN), lambda i: (jnp.minimum(i, _NB - 1), 0)),
            pl.BlockSpec((_E, _HIDDEN), lambda i: (0, 0)),
        ],
        out_specs=[
            pl.BlockSpec((_BT, _K), lambda i: (jnp.maximum(i - 1, 0), 0)),
            pl.BlockSpec((_BT, _K), lambda i: (jnp.maximum(i - 1, 0), 0)),
            pl.BlockSpec((_BT, _E), lambda i: (jnp.minimum(i, _NB - 1), 0)),
        ],
        out_shape=[
            jax.ShapeDtypeStruct((tokens, _K), jnp.float32),
            jax.ShapeDtypeStruct((tokens, _K), jnp.int32),
            jax.ShapeDtypeStruct((tokens, _E), jnp.float32),
        ],
        scratch_shapes=[pltpu.VMEM((2, _BT, _E), jnp.float32)],
    )(x, W_gate)
    return topw, topi, logits


# speedup vs baseline: 3.3296x; 1.3622x over previous
import jax
import jax.numpy as jnp
from jax.experimental import pallas as pl
from jax.experimental.pallas import tpu as pltpu

_HIDDEN = 4096
_E = 64
_K = 8
_BT = 512
_NB = 8192 // _BT
_RC = 64


def _topk_block(logits, topw_ref, topi_ref):
    iota = jax.lax.broadcasted_iota(
        jnp.int32, (_RC, _E), 1).astype(jnp.float32)
    for c in range(_BT // _RC):
        l = logits[c * _RC:(c + 1) * _RC, :]
        m = jnp.max(l, axis=1, keepdims=True)
        e = jnp.exp(l - m)
        s = jnp.sum(e, axis=1, keepdims=True)
        vals = e / s
        tops = []
        idxs = []
        total = jnp.zeros((_RC, 1), jnp.float32)
        for _ in range(_K):
            mv = jnp.max(vals, axis=1, keepdims=True)
            ix = jnp.min(jnp.where(vals == mv, iota, float(_E)),
                         axis=1, keepdims=True)
            tops.append(mv)
            idxs.append(ix)
            total = total + mv
            vals = jnp.where(iota == ix, -jnp.inf, vals)
        for j in range(_K):
            topw_ref[c * _RC:(c + 1) * _RC, j:j + 1] = tops[j] / total
            topi_ref[c * _RC:(c + 1) * _RC, j:j + 1] = (
                idxs[j].astype(jnp.int32))


def _router_kernel(x_ref, w_ref, topw_ref, topi_ref, logits_ref, lbuf):
    i = pl.program_id(0)

    logits = jax.lax.dot_general(
        x_ref[...], w_ref[...],
        dimension_numbers=(((1,), (1,)), ((), ())),
        preferred_element_type=jnp.float32,
    )
    logits_ref[...] = logits

    prev = lbuf[(i - 1) % 2]
    _topk_block(prev, topw_ref, topi_ref)
    lbuf[i % 2] = logits


@jax.jit
def kernel(x, W_gate):
    tokens = x.shape[0]
    topw, topi, logits = pl.pallas_call(
        _router_kernel,
        grid=(_NB + 1,),
        in_specs=[
            pl.BlockSpec((_BT, _HIDDEN), lambda i: (jnp.minimum(i, _NB - 1), 0)),
            pl.BlockSpec((_E, _HIDDEN), lambda i: (0, 0)),
        ],
        out_specs=[
            pl.BlockSpec((_BT, _K), lambda i: (jnp.maximum(i - 1, 0), 0)),
            pl.BlockSpec((_BT, _K), lambda i: (jnp.maximum(i - 1, 0), 0)),
            pl.BlockSpec((_BT, _E), lambda i: (jnp.minimum(i, _NB - 1), 0)),
        ],
        out_shape=[
            jax.ShapeDtypeStruct((tokens, _K), jnp.float32),
            jax.ShapeDtypeStruct((tokens, _K), jnp.int32),
            jax.ShapeDtypeStruct((tokens, _E), jnp.float32),
        ],
        scratch_shapes=[pltpu.VMEM((2, _BT, _E), jnp.float32)],
    )(x, W_gate)
    return topw, topi, logits
